# Initial kernel scaffold; baseline (speedup 1.0000x reference)
import jax
import jax.numpy as jnp
from jax.experimental import pallas as pl


def kernel(node_embeddings, edge_index, relation_type, relation_emb, is_symmetric):
    raise NotImplementedError("write your pallas kernel here")


# speedup vs baseline: 1.4569x; 1.4569x over previous
import functools

import numpy as np

import jax
import jax.numpy as jnp
from jax import lax
from jax.experimental import pallas as pl
from jax.experimental.pallas import tpu as pltpu
from jax.experimental.pallas import tpu_sc as plsc

EPS = 1e-08
D = 128
DH = 64
ROW = 144

def _normalize_body(x_ref, o_ref):
    x = x_ref[...]
    n2 = jnp.sum(x * x, axis=1, keepdims=True)
    scale = 1.0 / jnp.maximum(jnp.sqrt(n2), 1e-12)
    xn = x * scale
    nsq = n2 * (scale * scale)
    o_ref[...] = jnp.concatenate(
        [xn, jnp.broadcast_to(nsq, (x.shape[0], ROW - D))], axis=1)


def _normalize_table(node_embeddings):
    n = node_embeddings.shape[0]
    rows = 1000 if n % 1000 == 0 else n
    grid = n // rows
    return pl.pallas_call(
        _normalize_body,
        grid=(grid,),
        in_specs=[pl.BlockSpec((rows, D), lambda i: (i, 0))],
        out_specs=pl.BlockSpec((rows, ROW), lambda i: (i, 0)),
        out_shape=jax.ShapeDtypeStruct((n, ROW), jnp.float32),
    )(node_embeddings)


def _make_sc_kernel(n_edges, ch):
    info = plsc.get_sparse_core_info()
    nw = info.num_cores * info.num_subcores
    ew = n_edges // nw
    nchunks = ew // ch

    mesh = plsc.VectorSubcoreMesh(core_axis_name="c", subcore_axis_name="s")

    @functools.partial(
        pl.kernel,
        mesh=mesh,
        compiler_params=pltpu.CompilerParams(use_tc_tiling_on_sc=False),
        out_type=jax.ShapeDtypeStruct((n_edges, 32), jnp.float32),
        scratch_types=[
            pltpu.VMEM((ew,), jnp.int32),
            pltpu.VMEM((ew,), jnp.int32),
            pltpu.VMEM((ew,), jnp.int32),
            pltpu.VMEM((ch, ROW), jnp.float32),
            pltpu.VMEM((ch, ROW), jnp.float32),
            pltpu.VMEM((ch, ROW), jnp.float32),
            pltpu.VMEM((ch, 32), jnp.float32),
            pltpu.SemaphoreType.DMA,
        ],
    )
    def sc_kernel(table_hbm, w_hbm, sid_hbm, tid_hbm, rel_hbm,
                  out_hbm, sid_v, tid_v, rel_v, src_v, tgt_v, w_v,
                  pout_v, sem):
        wid = lax.axis_index("s") * info.num_cores + lax.axis_index("c")
        base = wid * ew
        pltpu.sync_copy(sid_hbm.at[pl.ds(base, ew)], sid_v)
        pltpu.sync_copy(tid_hbm.at[pl.ds(base, ew)], tid_v)
        pltpu.sync_copy(rel_hbm.at[pl.ds(base, ew)], rel_v)

        def chunk_body(c, carry):
            d1 = pltpu.async_copy(
                table_hbm.at[sid_v.at[pl.ds(c * ch, ch)]], src_v, sem)
            d2 = pltpu.async_copy(
                table_hbm.at[tid_v.at[pl.ds(c * ch, ch)]], tgt_v, sem)
            d3 = pltpu.async_copy(
                w_hbm.at[rel_v.at[pl.ds(c * ch, ch)]], w_v, sem)
            d1.wait()
            d2.wait()
            d3.wait()

            def edge_body(e, carry2):
                acc = jnp.zeros((16,), jnp.float32)
                for k in range(4):
                    hre = src_v[e, pl.ds(k * 16, 16)]
                    him = src_v[e, pl.ds(DH + k * 16, 16)]
                    tre = tgt_v[e, pl.ds(k * 16, 16)]
                    tim = tgt_v[e, pl.ds(DH + k * 16, 16)]
                    wc = w_v[e, pl.ds(k * 16, 16)]
                    ws = w_v[e, pl.ds(DH + k * 16, 16)]
                    a = hre * tre + him * tim
                    b = hre * tim - him * tre
                    acc = acc + wc * a + ws * b
                nh = src_v[e, pl.ds(D, 16)]
                nt = tgt_v[e, pl.ds(D, 16)]
                symf = w_v[e, pl.ds(D, 16)]
                misc = (nh + nt + 1.0) * (1.0 - 2.0 * symf)
                pout_v[e, pl.ds(0, 16)] = acc
                pout_v[e, pl.ds(16, 16)] = misc
                return carry2

            lax.fori_loop(0, ch, edge_body, 0)
            pltpu.sync_copy(pout_v, out_hbm.at[pl.ds(base + c * ch, ch)])
            return carry

        lax.fori_loop(0, nchunks, chunk_body, 0)

    return sc_kernel


def _reduce_body(p_ref, o_ref):
    p = p_ref[...]
    inner = jnp.sum(p[:, 0:16], axis=1)
    misc = p[:, 16]
    sym = misc < 0.0
    nhnt = jnp.abs(misc) - 1.0
    sq = jnp.maximum(nhnt - 2.0 * inner, 0.0) + EPS
    score = jnp.where(sym, inner * (1.0 / D) - 1.0, -jnp.sqrt(sq))
    o_ref[...] = score.reshape(o_ref.shape)


def _reduce_scores(partials, n_edges):
    lanes = 160
    rows = n_edges // lanes
    grid = 25
    br = rows // grid
    be = br * lanes
    out = pl.pallas_call(
        _reduce_body,
        grid=(grid,),
        in_specs=[pl.BlockSpec((be, 32), lambda i: (i, 0))],
        out_specs=pl.BlockSpec((br, lanes), lambda i: (i, 0)),
        out_shape=jax.ShapeDtypeStruct((rows, lanes), jnp.float32),
    )(partials)
    return out.reshape(n_edges)


def kernel(node_embeddings, edge_index, relation_type, relation_emb,
           is_symmetric):
    n_edges = edge_index.shape[1]
    ch = 80

    table = _normalize_table(node_embeddings)
    wc = jnp.where(is_symmetric[:, None], 1.0, jnp.cos(relation_emb))
    ws = jnp.where(is_symmetric[:, None], 0.0, jnp.sin(relation_emb))
    symf = jnp.broadcast_to(
        is_symmetric[:, None].astype(jnp.float32),
        (is_symmetric.shape[0], ROW - D))
    w = jnp.concatenate([wc, ws, symf], axis=1).astype(jnp.float32)

    sck = _make_sc_kernel(n_edges, ch)
    partials = sck(table, w, edge_index[0], edge_index[1], relation_type)
    return _reduce_scores(partials, n_edges)


# speedup vs baseline: 1.4588x; 1.0013x over previous
import functools

import numpy as np

import jax
import jax.numpy as jnp
from jax import lax
from jax.experimental import pallas as pl
from jax.experimental.pallas import tpu as pltpu
from jax.experimental.pallas import tpu_sc as plsc

EPS = 1e-08
D = 128
DH = 64
ROW = 144

def _normalize_body(x_ref, o_ref):
    x = x_ref[...]
    n2 = jnp.sum(x * x, axis=1, keepdims=True)
    scale = 1.0 / jnp.maximum(jnp.sqrt(n2), 1e-12)
    xn = x * scale
    nsq = n2 * (scale * scale)
    o_ref[...] = jnp.concatenate(
        [xn, jnp.broadcast_to(nsq, (x.shape[0], ROW - D))], axis=1)


def _normalize_table(node_embeddings):
    n = node_embeddings.shape[0]
    rows = 1000 if n % 1000 == 0 else n
    grid = n // rows
    return pl.pallas_call(
        _normalize_body,
        grid=(grid,),
        in_specs=[pl.BlockSpec((rows, D), lambda i: (i, 0))],
        out_specs=pl.BlockSpec((rows, ROW), lambda i: (i, 0)),
        out_shape=jax.ShapeDtypeStruct((n, ROW), jnp.float32),
    )(node_embeddings)


def _make_sc_kernel(n_edges, ch):
    info = plsc.get_sparse_core_info()
    nw = info.num_cores * info.num_subcores
    ew = n_edges // nw
    nchunks = ew // ch

    mesh = plsc.VectorSubcoreMesh(core_axis_name="c", subcore_axis_name="s")

    @functools.partial(
        pl.kernel,
        mesh=mesh,
        compiler_params=pltpu.CompilerParams(use_tc_tiling_on_sc=False),
        out_type=jax.ShapeDtypeStruct((n_edges, 32), jnp.float32),
        scratch_types=[
            pltpu.VMEM((ew,), jnp.int32),
            pltpu.VMEM((ew,), jnp.int32),
            pltpu.VMEM((ew,), jnp.int32),
            pltpu.VMEM((ch, ROW), jnp.float32),
            pltpu.VMEM((ch, ROW), jnp.float32),
            pltpu.VMEM((ch, ROW), jnp.float32),
            pltpu.VMEM((ch, 32), jnp.float32),
            pltpu.SemaphoreType.DMA,
        ],
    )
    def sc_kernel(table_hbm, w_hbm, sid_hbm, tid_hbm, rel_hbm,
                  out_hbm, sid_v, tid_v, rel_v, src_v, tgt_v, w_v,
                  pout_v, sem):
        wid = lax.axis_index("s") * info.num_cores + lax.axis_index("c")
        base = wid * ew
        pltpu.sync_copy(sid_hbm.at[pl.ds(base, ew)], sid_v)
        pltpu.sync_copy(tid_hbm.at[pl.ds(base, ew)], tid_v)
        pltpu.sync_copy(rel_hbm.at[pl.ds(base, ew)], rel_v)

        def chunk_body(c, carry):
            d1 = pltpu.async_copy(
                table_hbm.at[sid_v.at[pl.ds(c * ch, ch)]], src_v, sem)
            d2 = pltpu.async_copy(
                table_hbm.at[tid_v.at[pl.ds(c * ch, ch)]], tgt_v, sem)
            d3 = pltpu.async_copy(
                w_hbm.at[rel_v.at[pl.ds(c * ch, ch)]], w_v, sem)
            d1.wait()
            d2.wait()
            d3.wait()

            @plsc.parallel_loop(0, ch, 1, unroll=8)
            def edge_body(e):
                acc = jnp.zeros((16,), jnp.float32)
                for k in range(4):
                    hre = src_v[e, pl.ds(k * 16, 16)]
                    him = src_v[e, pl.ds(DH + k * 16, 16)]
                    tre = tgt_v[e, pl.ds(k * 16, 16)]
                    tim = tgt_v[e, pl.ds(DH + k * 16, 16)]
                    wc = w_v[e, pl.ds(k * 16, 16)]
                    ws = w_v[e, pl.ds(DH + k * 16, 16)]
                    a = hre * tre + him * tim
                    b = hre * tim - him * tre
                    acc = acc + wc * a + ws * b
                nh = src_v[e, pl.ds(D, 16)]
                nt = tgt_v[e, pl.ds(D, 16)]
                symf = w_v[e, pl.ds(D, 16)]
                misc = (nh + nt + 1.0) * (1.0 - 2.0 * symf)
                pout_v[e, pl.ds(0, 16)] = acc
                pout_v[e, pl.ds(16, 16)] = misc

            pltpu.sync_copy(pout_v, out_hbm.at[pl.ds(base + c * ch, ch)])
            return carry

        lax.fori_loop(0, nchunks, chunk_body, 0)

    return sc_kernel


def _reduce_body(p_ref, o_ref):
    p = p_ref[...]
    inner = jnp.sum(p[:, 0:16], axis=1)
    misc = p[:, 16]
    sym = misc < 0.0
    nhnt = jnp.abs(misc) - 1.0
    sq = jnp.maximum(nhnt - 2.0 * inner, 0.0) + EPS
    score = jnp.where(sym, inner * (1.0 / D) - 1.0, -jnp.sqrt(sq))
    o_ref[...] = score.reshape(o_ref.shape)


def _reduce_scores(partials, n_edges):
    lanes = 160
    rows = n_edges // lanes
    grid = 25
    br = rows // grid
    be = br * lanes
    out = pl.pallas_call(
        _reduce_body,
        grid=(grid,),
        in_specs=[pl.BlockSpec((be, 32), lambda i: (i, 0))],
        out_specs=pl.BlockSpec((br, lanes), lambda i: (i, 0)),
        out_shape=jax.ShapeDtypeStruct((rows, lanes), jnp.float32),
    )(partials)
    return out.reshape(n_edges)


def kernel(node_embeddings, edge_index, relation_type, relation_emb,
           is_symmetric):
    n_edges = edge_index.shape[1]
    ch = 80

    table = _normalize_table(node_embeddings)
    wc = jnp.where(is_symmetric[:, None], 1.0, jnp.cos(relation_emb))
    ws = jnp.where(is_symmetric[:, None], 0.0, jnp.sin(relation_emb))
    symf = jnp.broadcast_to(
        is_symmetric[:, None].astype(jnp.float32),
        (is_symmetric.shape[0], ROW - D))
    w = jnp.concatenate([wc, ws, symf], axis=1).astype(jnp.float32)

    sck = _make_sc_kernel(n_edges, ch)
    partials = sck(table, w, edge_index[0], edge_index[1], relation_type)
    return _reduce_scores(partials, n_edges)


# speedup vs baseline: 4.8709x; 3.3389x over previous
import functools

import numpy as np

import jax
import jax.numpy as jnp
from jax import lax
from jax.experimental import pallas as pl
from jax.experimental.pallas import tpu as pltpu
from jax.experimental.pallas import tpu_sc as plsc

EPS = 1e-08
D = 128
DH = 64
ROWB = 160
ROWW = 144

_PERM = np.concatenate([np.arange(0, 32, 2), np.arange(1, 32, 2),
                        np.arange(32, 64, 2), np.arange(33, 64, 2)])


def _normalize_body(x_ref, o_ref):
    x = x_ref[...]
    n2 = jnp.sum(x * x, axis=1, keepdims=True)
    scale = 1.0 / jnp.maximum(jnp.sqrt(n2), 1e-12)
    xn = x * scale
    nsq = n2 * (scale * scale)
    o_ref[...] = jnp.concatenate(
        [xn, jnp.broadcast_to(nsq, (x.shape[0], ROWB - D))],
        axis=1).astype(jnp.bfloat16)


def _normalize_table(node_embeddings):
    n = node_embeddings.shape[0]
    rows = 1000 if n % 1000 == 0 else n
    grid = n // rows
    return pl.pallas_call(
        _normalize_body,
        grid=(grid,),
        in_specs=[pl.BlockSpec((rows, D), lambda i: (i, 0))],
        out_specs=pl.BlockSpec((rows, ROWB), lambda i: (i, 0)),
        out_shape=jax.ShapeDtypeStruct((n, ROWB), jnp.bfloat16),
    )(node_embeddings)


def _make_sc_kernel(n_edges, n_nodes, ch):
    info = plsc.get_sparse_core_info()
    nw = info.num_cores * info.num_subcores
    ew = n_edges // nw
    nchunks = ew // ch
    slab_step = n_nodes // info.num_subcores
    slab_len = (slab_step + 7) // 8 * 8 + 8

    mesh = plsc.VectorSubcoreMesh(core_axis_name="c", subcore_axis_name="s")

    @functools.partial(
        pl.kernel,
        mesh=mesh,
        compiler_params=pltpu.CompilerParams(use_tc_tiling_on_sc=False,
                                             needs_layout_passes=False),
        out_type=jax.ShapeDtypeStruct((n_edges, 32), jnp.float32),
        scratch_types=[
            pltpu.VMEM((ew,), jnp.int32),
            pltpu.VMEM((ew,), jnp.int32),
            pltpu.VMEM((ew + 16,), jnp.int32),
            pltpu.VMEM((8, ROWW), jnp.float32),
            pltpu.VMEM((ch, ROWB), jnp.bfloat16),
            pltpu.VMEM((ch, ROWB), jnp.bfloat16),
            pltpu.VMEM((ch, 32), jnp.float32),
            pltpu.VMEM_SHARED((n_nodes, ROWB), jnp.bfloat16),
            pltpu.SemaphoreType.DMA,
        ],
    )
    def sc_kernel(table_hbm, w_hbm, sid_hbm, tid_hbm, rel_hbm,
                  out_hbm, sid_v, tid_v, rel_v, w_v, src_v, tgt_v,
                  pout_v, table_sh, sem):
        cid = lax.axis_index("c")
        sid = lax.axis_index("s")
        wid = sid * info.num_cores + cid
        base = wid * ew
        slab = pl.multiple_of((sid * slab_step) // 8 * 8, 8)
        pltpu.sync_copy(table_hbm.at[pl.ds(slab, slab_len)],
                        table_sh.at[pl.ds(slab, slab_len)])
        pltpu.sync_copy(w_hbm, w_v)
        pltpu.sync_copy(sid_hbm.at[pl.ds(base, ew)], sid_v)
        pltpu.sync_copy(tid_hbm.at[pl.ds(base, ew)], tid_v)
        pltpu.sync_copy(rel_hbm.at[pl.ds(base, ew)], rel_v.at[pl.ds(0, ew)])
        plsc.subcore_barrier()

        def chunk_body(c, carry):
            d1 = pltpu.async_copy(
                table_sh.at[sid_v.at[pl.ds(c * ch, ch)]], src_v, sem)
            d2 = pltpu.async_copy(
                table_sh.at[tid_v.at[pl.ds(c * ch, ch)]], tgt_v, sem)
            d1.wait()
            d2.wait()

            @plsc.parallel_loop(0, ch, 1, unroll=8)
            def edge_body(e):
                r = rel_v[pl.ds(c * ch + e, 16)][0]
                acc = jnp.zeros((16,), jnp.float32)
                for kk in range(2):
                    sre = plsc.unpack(src_v[e, pl.ds(32 * kk, 32)],
                                      format=plsc.PackFormat.INTERLEAVED)
                    sim = plsc.unpack(src_v[e, pl.ds(DH + 32 * kk, 32)],
                                      format=plsc.PackFormat.INTERLEAVED)
                    tre = plsc.unpack(tgt_v[e, pl.ds(32 * kk, 32)],
                                      format=plsc.PackFormat.INTERLEAVED)
                    tim = plsc.unpack(tgt_v[e, pl.ds(DH + 32 * kk, 32)],
                                      format=plsc.PackFormat.INTERLEAVED)
                    for par in range(2):
                        wc = w_v[r, pl.ds(32 * kk + 16 * par, 16)]
                        ws = w_v[r, pl.ds(DH + 32 * kk + 16 * par, 16)]
                        a = sre[par] * tre[par] + sim[par] * tim[par]
                        b = sre[par] * tim[par] - sim[par] * tre[par]
                        acc = acc + wc * a + ws * b
                nh = plsc.unpack(src_v[e, pl.ds(D, 32)],
                                 format=plsc.PackFormat.INTERLEAVED)[0]
                nt = plsc.unpack(tgt_v[e, pl.ds(D, 32)],
                                 format=plsc.PackFormat.INTERLEAVED)[0]
                symf = w_v[r, pl.ds(D, 16)]
                misc = (nh + nt + 1.0) * (1.0 - 2.0 * symf)
                pout_v[e, pl.ds(0, 16)] = acc
                pout_v[e, pl.ds(16, 16)] = misc

            pltpu.sync_copy(pout_v, out_hbm.at[pl.ds(base + c * ch, ch)])
            return carry

        lax.fori_loop(0, nchunks, chunk_body, 0)

    return sc_kernel


def _reduce_body(p_ref, o_ref):
    p = p_ref[...]
    inner = jnp.sum(p[:, 0:16], axis=1)
    misc = p[:, 16]
    sym = misc < 0.0
    nhnt = jnp.abs(misc) - 1.0
    sq = jnp.maximum(nhnt - 2.0 * inner, 0.0) + EPS
    score = jnp.where(sym, inner * (1.0 / D) - 1.0, -jnp.sqrt(sq))
    o_ref[...] = score.reshape(o_ref.shape)


def _reduce_scores(partials, n_edges):
    lanes = 160
    rows = n_edges // lanes
    grid = 25
    br = rows // grid
    be = br * lanes
    out = pl.pallas_call(
        _reduce_body,
        grid=(grid,),
        in_specs=[pl.BlockSpec((be, 32), lambda i: (i, 0))],
        out_specs=pl.BlockSpec((br, lanes), lambda i: (i, 0)),
        out_shape=jax.ShapeDtypeStruct((rows, lanes), jnp.float32),
    )(partials)
    return out.reshape(n_edges)


def kernel(node_embeddings, edge_index, relation_type, relation_emb,
           is_symmetric):
    n_edges = edge_index.shape[1]
    n_nodes = node_embeddings.shape[0]
    ch = 80

    table = _normalize_table(node_embeddings)
    wc = jnp.where(is_symmetric[:, None], 1.0, jnp.cos(relation_emb))
    ws = jnp.where(is_symmetric[:, None], 0.0, jnp.sin(relation_emb))
    wcp = jnp.take(wc, _PERM, axis=1)
    wsp = jnp.take(ws, _PERM, axis=1)
    symf = jnp.broadcast_to(
        is_symmetric[:, None].astype(jnp.float32),
        (is_symmetric.shape[0], ROWW - D))
    w = jnp.concatenate([wcp, wsp, symf], axis=1).astype(jnp.float32)

    sck = _make_sc_kernel(n_edges, n_nodes, ch)
    partials = sck(table, w, edge_index[0], edge_index[1], relation_type)
    return _reduce_scores(partials, n_edges)


# speedup vs baseline: 10.9062x; 2.2391x over previous
import functools

import numpy as np

import jax
import jax.numpy as jnp
from jax import lax
from jax.experimental import pallas as pl
from jax.experimental.pallas import tpu as pltpu
from jax.experimental.pallas import tpu_sc as plsc

EPS = 1e-08
D = 128
DH = 64
ROWB = 160
ROWW = 144

_PERM = np.concatenate([np.arange(0, 32, 2), np.arange(1, 32, 2),
                        np.arange(32, 64, 2), np.arange(33, 64, 2)])


def _normalize_body(x_ref, o_ref):
    x = x_ref[...]
    n2 = jnp.sum(x * x, axis=1, keepdims=True)
    scale = 1.0 / jnp.maximum(jnp.sqrt(n2), 1e-12)
    xn = x * scale
    nsq = n2 * (scale * scale)
    o_ref[...] = jnp.concatenate(
        [xn, jnp.broadcast_to(nsq, (x.shape[0], ROWB - D))],
        axis=1).astype(jnp.bfloat16)


def _normalize_table(node_embeddings):
    n = node_embeddings.shape[0]
    rows = 1000 if n % 1000 == 0 else n
    grid = n // rows
    return pl.pallas_call(
        _normalize_body,
        grid=(grid,),
        in_specs=[pl.BlockSpec((rows, D), lambda i: (i, 0))],
        out_specs=pl.BlockSpec((rows, ROWB), lambda i: (i, 0)),
        out_shape=jax.ShapeDtypeStruct((n, ROWB), jnp.bfloat16),
    )(node_embeddings)


def _make_sc_kernel(n_edges, n_nodes, ch):
    info = plsc.get_sparse_core_info()
    nw = info.num_cores * info.num_subcores
    ew = n_edges // nw
    nchunks = ew // ch
    slab_step = n_nodes // info.num_subcores
    slab_len = (slab_step + 7) // 8 * 8 + 8

    mesh = plsc.VectorSubcoreMesh(core_axis_name="c", subcore_axis_name="s")

    @functools.partial(
        pl.kernel,
        mesh=mesh,
        compiler_params=pltpu.CompilerParams(use_tc_tiling_on_sc=False,
                                             needs_layout_passes=False),
        out_type=jax.ShapeDtypeStruct((n_edges,), jnp.float32),
        scratch_types=[
            pltpu.VMEM((ew,), jnp.int32),
            pltpu.VMEM((ew,), jnp.int32),
            pltpu.VMEM((ew + 16,), jnp.int32),
            pltpu.VMEM((8, ROWW), jnp.float32),
            pltpu.VMEM((ch, ROWB), jnp.bfloat16),
            pltpu.VMEM((ch, ROWB), jnp.bfloat16),
            pltpu.VMEM((ew,), jnp.float32),
            pltpu.VMEM_SHARED((n_nodes, ROWB), jnp.bfloat16),
            pltpu.SemaphoreType.DMA,
        ],
    )
    def sc_kernel(table_hbm, w_hbm, sid_hbm, tid_hbm, rel_hbm,
                  out_hbm, sid_v, tid_v, rel_v, w_v, src_v, tgt_v,
                  score_v, table_sh, sem):
        cid = lax.axis_index("c")
        sid = lax.axis_index("s")
        wid = sid * info.num_cores + cid
        base = wid * ew
        slab = pl.multiple_of((sid * slab_step) // 8 * 8, 8)
        pltpu.sync_copy(table_hbm.at[pl.ds(slab, slab_len)],
                        table_sh.at[pl.ds(slab, slab_len)])
        pltpu.sync_copy(w_hbm, w_v)
        pltpu.sync_copy(sid_hbm.at[pl.ds(base, ew)], sid_v)
        pltpu.sync_copy(tid_hbm.at[pl.ds(base, ew)], tid_v)
        pltpu.sync_copy(rel_hbm.at[pl.ds(base, ew)], rel_v.at[pl.ds(0, ew)])
        plsc.subcore_barrier()

        def chunk_body(c, carry):
            d1 = pltpu.async_copy(
                table_sh.at[sid_v.at[pl.ds(c * ch, ch)]], src_v, sem)
            d2 = pltpu.async_copy(
                table_sh.at[tid_v.at[pl.ds(c * ch, ch)]], tgt_v, sem)
            d1.wait()
            d2.wait()
            lane = lax.iota(jnp.int32, 16)
            magic = jnp.full((16,), 0x5F3759DF, jnp.int32)

            @plsc.parallel_loop(0, ch // 16, 1, unroll=1)
            def group_body(g):
                score_vec = jnp.zeros((16,), jnp.float32)
                inner_vec = jnp.zeros((16,), jnp.float32)
                nh_vec = jnp.zeros((16,), jnp.float32)
                nt_vec = jnp.zeros((16,), jnp.float32)
                sym_vec = jnp.zeros((16,), jnp.float32)
                for i in range(16):
                    e = g * 16 + i
                    r = rel_v[pl.ds(c * ch + e, 16)][0]
                    acc = jnp.zeros((16,), jnp.float32)
                    for kk in range(2):
                        sre = plsc.unpack(src_v[e, pl.ds(32 * kk, 32)],
                                          format=plsc.PackFormat.INTERLEAVED)
                        sim = plsc.unpack(src_v[e, pl.ds(DH + 32 * kk, 32)],
                                          format=plsc.PackFormat.INTERLEAVED)
                        tre = plsc.unpack(tgt_v[e, pl.ds(32 * kk, 32)],
                                          format=plsc.PackFormat.INTERLEAVED)
                        tim = plsc.unpack(tgt_v[e, pl.ds(DH + 32 * kk, 32)],
                                          format=plsc.PackFormat.INTERLEAVED)
                        for par in range(2):
                            wc = w_v[r, pl.ds(32 * kk + 16 * par, 16)]
                            ws = w_v[r, pl.ds(DH + 32 * kk + 16 * par, 16)]
                            a = sre[par] * tre[par] + sim[par] * tim[par]
                            b = sre[par] * tim[par] - sim[par] * tre[par]
                            acc = acc + wc * a + ws * b
                    nh = plsc.unpack(src_v[e, pl.ds(D, 32)],
                                     format=plsc.PackFormat.INTERLEAVED)[0]
                    nt = plsc.unpack(tgt_v[e, pl.ds(D, 32)],
                                     format=plsc.PackFormat.INTERLEAVED)[0]
                    symf = w_v[r, pl.ds(D, 16)]
                    inner = jnp.full((16,), jnp.sum(acc))
                    sel = lane == i
                    inner_vec = jnp.where(sel, inner, inner_vec)
                    nh_vec = jnp.where(sel, nh, nh_vec)
                    nt_vec = jnp.where(sel, nt, nt_vec)
                    sym_vec = jnp.where(sel, symf, sym_vec)
                sq = jnp.maximum(nh_vec + nt_vec - 2.0 * inner_vec, 0.0) + EPS
                y = lax.bitcast_convert_type(
                    magic - lax.shift_right_arithmetic(
                        lax.bitcast_convert_type(sq, jnp.int32), 1),
                    jnp.float32)
                for _ in range(3):
                    y = y * (1.5 - 0.5 * sq * y * y)
                score_vec = jnp.where(sym_vec > 0.5,
                                      inner_vec * (1.0 / D) - 1.0,
                                      -sq * y)
                score_v[pl.ds(c * ch + g * 16, 16)] = score_vec

            return carry

        lax.fori_loop(0, nchunks, chunk_body, 0)
        pltpu.sync_copy(score_v, out_hbm.at[pl.ds(base, ew)])

    return sc_kernel


def kernel(node_embeddings, edge_index, relation_type, relation_emb,
           is_symmetric):
    n_edges = edge_index.shape[1]
    n_nodes = node_embeddings.shape[0]
    ch = 80

    table = _normalize_table(node_embeddings)
    wc = jnp.where(is_symmetric[:, None], 1.0, jnp.cos(relation_emb))
    ws = jnp.where(is_symmetric[:, None], 0.0, jnp.sin(relation_emb))
    wcp = jnp.take(wc, _PERM, axis=1)
    wsp = jnp.take(ws, _PERM, axis=1)
    symf = jnp.broadcast_to(
        is_symmetric[:, None].astype(jnp.float32),
        (is_symmetric.shape[0], ROWW - D))
    w = jnp.concatenate([wcp, wsp, symf], axis=1).astype(jnp.float32)

    sck = _make_sc_kernel(n_edges, n_nodes, ch)
    return sck(table, w, edge_index[0], edge_index[1], relation_type)


# speedup vs baseline: 14.4045x; 1.3208x over previous
import functools

import numpy as np

import jax
import jax.numpy as jnp
from jax import lax
from jax.experimental import pallas as pl
from jax.experimental.pallas import tpu as pltpu
from jax.experimental.pallas import tpu_sc as plsc

EPS = 1e-08
D = 128
DH = 64
ROWB = 160
ROWW = 144

_PERM = np.concatenate([np.arange(0, 32, 2), np.arange(1, 32, 2),
                        np.arange(32, 64, 2), np.arange(33, 64, 2)])


def _normalize_body(x_ref, o_ref):
    x = x_ref[...]
    n2 = jnp.sum(x * x, axis=1, keepdims=True)
    scale = 1.0 / jnp.maximum(jnp.sqrt(n2), 1e-12)
    xn = x * scale
    nsq = n2 * (scale * scale)
    o_ref[...] = jnp.concatenate(
        [xn, jnp.broadcast_to(nsq, (x.shape[0], ROWB - D))],
        axis=1).astype(jnp.bfloat16)


def _normalize_table(node_embeddings):
    n = node_embeddings.shape[0]
    rows = 1000 if n % 1000 == 0 else n
    grid = n // rows
    return pl.pallas_call(
        _normalize_body,
        grid=(grid,),
        in_specs=[pl.BlockSpec((rows, D), lambda i: (i, 0))],
        out_specs=pl.BlockSpec((rows, ROWB), lambda i: (i, 0)),
        out_shape=jax.ShapeDtypeStruct((n, ROWB), jnp.bfloat16),
    )(node_embeddings)


def _make_sc_kernel(n_edges, n_nodes, ch):
    info = plsc.get_sparse_core_info()
    nw = info.num_cores * info.num_subcores
    ew = n_edges // nw
    nchunks = ew // ch
    slab_step = n_nodes // info.num_subcores
    slab_len = (slab_step + 7) // 8 * 8 + 8

    mesh = plsc.VectorSubcoreMesh(core_axis_name="c", subcore_axis_name="s")

    @functools.partial(
        pl.kernel,
        mesh=mesh,
        compiler_params=pltpu.CompilerParams(use_tc_tiling_on_sc=False,
                                             needs_layout_passes=False),
        out_type=jax.ShapeDtypeStruct((n_edges,), jnp.float32),
        scratch_types=[
            pltpu.VMEM((ew,), jnp.int32),
            pltpu.VMEM((ew,), jnp.int32),
            pltpu.VMEM((ew + 16,), jnp.int32),
            pltpu.VMEM((8, ROWW), jnp.float32),
            pltpu.VMEM((ch, ROWB), jnp.bfloat16),
            pltpu.VMEM((ch, ROWB), jnp.bfloat16),
            pltpu.VMEM((ch, ROWB), jnp.bfloat16),
            pltpu.VMEM((ch, ROWB), jnp.bfloat16),
            pltpu.VMEM((ew,), jnp.float32),
            pltpu.VMEM_SHARED((n_nodes, ROWB), jnp.bfloat16),
            pltpu.SemaphoreType.DMA,
            pltpu.SemaphoreType.DMA,
        ],
    )
    def sc_kernel(table_hbm, w_hbm, sid_hbm, tid_hbm, rel_hbm,
                  out_hbm, sid_v, tid_v, rel_v, w_v, src0_v, tgt0_v,
                  src1_v, tgt1_v, score_v, table_sh, sem0, sem1):
        cid = lax.axis_index("c")
        sid = lax.axis_index("s")
        wid = sid * info.num_cores + cid
        base = wid * ew
        slab = pl.multiple_of((sid * slab_step) // 8 * 8, 8)
        pltpu.sync_copy(table_hbm.at[pl.ds(slab, slab_len)],
                        table_sh.at[pl.ds(slab, slab_len)])
        pltpu.sync_copy(w_hbm, w_v)
        pltpu.sync_copy(sid_hbm.at[pl.ds(base, ew)], sid_v)
        pltpu.sync_copy(tid_hbm.at[pl.ds(base, ew)], tid_v)
        pltpu.sync_copy(rel_hbm.at[pl.ds(base, ew)], rel_v.at[pl.ds(0, ew)])
        plsc.subcore_barrier()

        lane = lax.iota(jnp.int32, 16)
        magic = jnp.full((16,), 0x5F3759DF, jnp.int32)

        def issue(c, sv, tv, gs):
            pltpu.async_copy(table_sh.at[sid_v.at[pl.ds(c * ch, ch)]], sv, gs)
            pltpu.async_copy(table_sh.at[tid_v.at[pl.ds(c * ch, ch)]], tv, gs)

        def drain(sv, tv, gs):
            pltpu.make_async_copy(table_hbm.at[pl.ds(0, ch)], sv, gs).wait()
            pltpu.make_async_copy(table_hbm.at[pl.ds(0, ch)], tv, gs).wait()

        def compute(c, src_v, tgt_v):
            @plsc.parallel_loop(0, ch // 16, 1, unroll=1)
            def group_body(g):
                score_vec = jnp.zeros((16,), jnp.float32)
                inner_vec = jnp.zeros((16,), jnp.float32)
                nh_vec = jnp.zeros((16,), jnp.float32)
                nt_vec = jnp.zeros((16,), jnp.float32)
                sym_vec = jnp.zeros((16,), jnp.float32)
                for i in range(16):
                    e = g * 16 + i
                    r = rel_v[pl.ds(c * ch + e, 16)][0]
                    acc = jnp.zeros((16,), jnp.float32)
                    for kk in range(2):
                        sre = plsc.unpack(src_v[e, pl.ds(32 * kk, 32)],
                                          format=plsc.PackFormat.INTERLEAVED)
                        sim = plsc.unpack(src_v[e, pl.ds(DH + 32 * kk, 32)],
                                          format=plsc.PackFormat.INTERLEAVED)
                        tre = plsc.unpack(tgt_v[e, pl.ds(32 * kk, 32)],
                                          format=plsc.PackFormat.INTERLEAVED)
                        tim = plsc.unpack(tgt_v[e, pl.ds(DH + 32 * kk, 32)],
                                          format=plsc.PackFormat.INTERLEAVED)
                        for par in range(2):
                            wc = w_v[r, pl.ds(32 * kk + 16 * par, 16)]
                            ws = w_v[r, pl.ds(DH + 32 * kk + 16 * par, 16)]
                            a = sre[par] * tre[par] + sim[par] * tim[par]
                            b = sre[par] * tim[par] - sim[par] * tre[par]
                            acc = acc + wc * a + ws * b
                    nh = plsc.unpack(src_v[e, pl.ds(D, 32)],
                                     format=plsc.PackFormat.INTERLEAVED)[0]
                    nt = plsc.unpack(tgt_v[e, pl.ds(D, 32)],
                                     format=plsc.PackFormat.INTERLEAVED)[0]
                    symf = w_v[r, pl.ds(D, 16)]
                    inner = jnp.full((16,), jnp.sum(acc))
                    sel = lane == i
                    inner_vec = jnp.where(sel, inner, inner_vec)
                    nh_vec = jnp.where(sel, nh, nh_vec)
                    nt_vec = jnp.where(sel, nt, nt_vec)
                    sym_vec = jnp.where(sel, symf, sym_vec)
                sq = jnp.maximum(nh_vec + nt_vec - 2.0 * inner_vec, 0.0) + EPS
                y = lax.bitcast_convert_type(
                    magic - lax.shift_right_arithmetic(
                        lax.bitcast_convert_type(sq, jnp.int32), 1),
                    jnp.float32)
                for _ in range(3):
                    y = y * (1.5 - 0.5 * sq * y * y)
                score_vec = jnp.where(sym_vec > 0.5,
                                      inner_vec * (1.0 / D) - 1.0,
                                      -sq * y)
                score_v[pl.ds(c * ch + g * 16, 16)] = score_vec

        issue(0, src0_v, tgt0_v, sem0)

        def pair_body(i, carry):
            c0 = 2 * i
            issue(c0 + 1, src1_v, tgt1_v, sem1)
            drain(src0_v, tgt0_v, sem0)
            compute(c0, src0_v, tgt0_v)
            issue(c0 + 2, src0_v, tgt0_v, sem0)
            drain(src1_v, tgt1_v, sem1)
            compute(c0 + 1, src1_v, tgt1_v)
            return carry

        lax.fori_loop(0, (nchunks - 1) // 2, pair_body, 0)
        drain(src0_v, tgt0_v, sem0)
        compute(nchunks - 1, src0_v, tgt0_v)
        pltpu.sync_copy(score_v, out_hbm.at[pl.ds(base, ew)])

    return sc_kernel


def kernel(node_embeddings, edge_index, relation_type, relation_emb,
           is_symmetric):
    n_edges = edge_index.shape[1]
    n_nodes = node_embeddings.shape[0]
    ch = 80

    table = _normalize_table(node_embeddings)
    wc = jnp.where(is_symmetric[:, None], 1.0, jnp.cos(relation_emb))
    ws = jnp.where(is_symmetric[:, None], 0.0, jnp.sin(relation_emb))
    wcp = jnp.take(wc, _PERM, axis=1)
    wsp = jnp.take(ws, _PERM, axis=1)
    symf = jnp.broadcast_to(
        is_symmetric[:, None].astype(jnp.float32),
        (is_symmetric.shape[0], ROWW - D))
    w = jnp.concatenate([wcp, wsp, symf], axis=1).astype(jnp.float32)

    sck = _make_sc_kernel(n_edges, n_nodes, ch)
    return sck(table, w, edge_index[0], edge_index[1], relation_type)
